# Initial kernel scaffold; baseline (speedup 1.0000x reference)
import jax
import jax.numpy as jnp
from jax.experimental import pallas as pl


def kernel(x, u_table, m_table, fc_w, fc_b):
    raise NotImplementedError("write your pallas kernel here")


# speedup vs baseline: 1.0349x; 1.0349x over previous
import functools

import jax
import jax.numpy as jnp
from jax import lax
from jax.experimental import pallas as pl
from jax.experimental.pallas import tpu as pltpu
from jax.experimental.pallas import tpu_sc as plsc

_B = 16384
_NW = 32
_BPW = _B // _NW
_CH = 128
_NCH = _BPW // _CH
_ROWS = _B // _CH
_L = 16


def _make_sc_kernel():
    mesh = plsc.VectorSubcoreMesh(core_axis_name="c", subcore_axis_name="s")

    @functools.partial(
        pl.kernel,
        mesh=mesh,
        out_type=jax.ShapeDtypeStruct((_ROWS, _CH), jnp.float32),
        scratch_types=[
            pltpu.VMEM((_NCH, _CH), jnp.int32),
            pltpu.VMEM((_NCH, _CH), jnp.int32),
            pltpu.VMEM((_NCH, _CH), jnp.float32),
            pltpu.VMEM((_NCH, _CH), jnp.float32),
            pltpu.VMEM((_L,), jnp.float32),
            pltpu.VMEM((_L,), jnp.float32),
            pltpu.VMEM((_NCH, _CH), jnp.float32),
            pltpu.SemaphoreType.DMA,
        ],
    )
    def body(uid_hbm, mid_hbm, ut_hbm, mt_hbm, w_hbm, b_hbm, out_hbm,
             uid_v, mid_v, uv_v, mv_v, w_v, b_v, o_v, sem):
        wid = lax.axis_index("s") * 2 + lax.axis_index("c")
        r0 = wid * _NCH
        pltpu.sync_copy(uid_hbm.at[pl.ds(r0, _NCH)], uid_v)
        pltpu.sync_copy(mid_hbm.at[pl.ds(r0, _NCH)], mid_v)
        pltpu.sync_copy(w_hbm, w_v)
        pltpu.sync_copy(b_hbm, b_v)
        handles = []
        for j in range(_NCH):
            handles.append(pltpu.async_copy(ut_hbm.at[uid_v.at[j]], uv_v.at[j], sem))
            handles.append(pltpu.async_copy(mt_hbm.at[mid_v.at[j]], mv_v.at[j], sem))
        for h in handles:
            h.wait()
        w = w_v[...]
        b = b_v[...]
        for j in range(_NCH):
            for i in range(_CH // _L):
                sl = pl.ds(i * _L, _L)
                z = uv_v[j, sl] * mv_v[j, sl] * w + b
                o_v[j, sl] = 1.0 / (1.0 + jnp.exp(-z))
        pltpu.sync_copy(o_v, out_hbm.at[pl.ds(r0, _NCH)])

    return body


_SC_KERNEL = _make_sc_kernel()


def kernel(x, u_table, m_table, fc_w, fc_b):
    uid = x[:, 0].astype(jnp.int32).reshape(_ROWS, _CH)
    mid = x[:, 1].astype(jnp.int32).reshape(_ROWS, _CH)
    ut = u_table.reshape(-1)
    mt = m_table.reshape(-1)
    wv = jnp.broadcast_to(fc_w.reshape(1), (_L,)).astype(jnp.float32)
    bv = jnp.broadcast_to(fc_b.reshape(1), (_L,)).astype(jnp.float32)
    out = _SC_KERNEL(uid, mid, ut, mt, wv, bv)
    return out.reshape(_B, 1)


# speedup vs baseline: 4.4668x; 4.3161x over previous
import functools

import jax
import jax.numpy as jnp
from jax import lax
from jax.experimental import pallas as pl
from jax.experimental.pallas import tpu as pltpu
from jax.experimental.pallas import tpu_sc as plsc

_B = 16384
_NW = 32
_BPW = _B // _NW
_CH = 128
_NCH = _BPW // _CH
_ROWS = _B // _CH
_L = 16


def _make_sc_kernel():
    mesh = plsc.VectorSubcoreMesh(core_axis_name="c", subcore_axis_name="s")

    @functools.partial(
        pl.kernel,
        mesh=mesh,
        out_type=jax.ShapeDtypeStruct((_ROWS, _CH), jnp.float32),
        scratch_types=[
            pltpu.VMEM((_NCH, _CH), jnp.int32),
            pltpu.VMEM((_NCH, _CH), jnp.int32),
            pltpu.VMEM((_NCH, _CH), jnp.float32),
            pltpu.VMEM((_NCH, _CH), jnp.float32),
            pltpu.VMEM((_L,), jnp.float32),
            pltpu.VMEM((_L,), jnp.float32),
            pltpu.VMEM((_NCH, _CH), jnp.float32),
            pltpu.SemaphoreType.DMA,
        ],
    )
    def body(uid_hbm, mid_hbm, ut_hbm, mt_hbm, w_hbm, b_hbm, out_hbm,
             uid_v, mid_v, uv_v, mv_v, w_v, b_v, o_v, sem):
        wid = lax.axis_index("s") * 2 + lax.axis_index("c")
        r0 = wid * _NCH
        pltpu.sync_copy(uid_hbm.at[pl.ds(r0, _NCH)], uid_v)
        pltpu.sync_copy(mid_hbm.at[pl.ds(r0, _NCH)], mid_v)
        pltpu.sync_copy(w_hbm, w_v)
        pltpu.sync_copy(b_hbm, b_v)
        ut = ut_hbm.at[0]
        mt = mt_hbm.at[0]
        handles = []
        for j in range(_NCH):
            handles.append(pltpu.async_copy(ut.at[uid_v.at[j]], uv_v.at[j], sem))
            handles.append(pltpu.async_copy(mt.at[mid_v.at[j]], mv_v.at[j], sem))
        for h in handles:
            h.wait()
        w = w_v[...]
        b = b_v[...]
        for j in range(_NCH):
            for i in range(_CH // _L):
                sl = pl.ds(i * _L, _L)
                z = uv_v[j, sl] * mv_v[j, sl] * w + b
                o_v[j, sl] = 1.0 / (1.0 + jnp.exp(-z))
        pltpu.sync_copy(o_v, out_hbm.at[pl.ds(r0, _NCH)])

    return body


_SC_KERNEL = _make_sc_kernel()


def kernel(x, u_table, m_table, fc_w, fc_b):
    uid = x[:, 0].astype(jnp.int32).reshape(_ROWS, _CH)
    mid = x[:, 1].astype(jnp.int32).reshape(_ROWS, _CH)
    wv = jnp.broadcast_to(fc_w.reshape(1), (_L,)).astype(jnp.float32)
    bv = jnp.broadcast_to(fc_b.reshape(1), (_L,)).astype(jnp.float32)
    out = _SC_KERNEL(uid, mid, u_table.reshape(1, -1), m_table.reshape(1, -1), wv, bv)
    return out.reshape(_B, 1)


# speedup vs baseline: 4.9295x; 1.1036x over previous
import functools

import jax
import jax.numpy as jnp
from jax import lax
from jax.experimental import pallas as pl
from jax.experimental.pallas import tpu as pltpu
from jax.experimental.pallas import tpu_sc as plsc

_B = 16384
_NW = 32
_BPW = _B // _NW
_CH = 128
_NCH = _BPW // _CH
_ROWS = _B // _CH
_L = 16


def _make_sc_kernel():
    mesh = plsc.VectorSubcoreMesh(core_axis_name="c", subcore_axis_name="s")

    @functools.partial(
        pl.kernel,
        mesh=mesh,
        out_type=jax.ShapeDtypeStruct((_ROWS, _CH), jnp.float32),
        scratch_types=[
            pltpu.VMEM((2 * _NCH, _CH), jnp.int32),
            pltpu.VMEM((_NCH, _CH), jnp.float32),
            pltpu.VMEM((_NCH, _CH), jnp.float32),
            pltpu.VMEM((2 * _L,), jnp.float32),
            pltpu.VMEM((_NCH, _CH), jnp.float32),
            pltpu.SemaphoreType.DMA,
        ],
    )
    def body(x_hbm, ut_hbm, mt_hbm, wb_hbm, out_hbm,
             idx_v, uv_v, mv_v, wb_v, o_v, sem):
        wid = lax.axis_index("s") * 2 + lax.axis_index("c")
        r0 = wid * _NCH
        pltpu.sync_copy(x_hbm.at[pl.ds(2 * r0, 2 * _NCH)], idx_v)
        pltpu.sync_copy(wb_hbm, wb_v)
        ut = ut_hbm.at[0]
        mt = mt_hbm.at[0]
        handles = []
        for j in range(_NCH):
            uid = idx_v.at[2 * j]
            mid = idx_v.at[2 * j + 1]
            handles.append(pltpu.async_copy(ut.at[uid], uv_v.at[j], sem))
            handles.append(pltpu.async_copy(mt.at[mid], mv_v.at[j], sem))
        w = wb_v[pl.ds(0, _L)]
        b = wb_v[pl.ds(_L, _L)]
        for j in range(_NCH):
            handles[2 * j].wait()
            handles[2 * j + 1].wait()
            for i in range(_CH // _L):
                sl = pl.ds(i * _L, _L)
                z = uv_v[j, sl] * mv_v[j, sl] * w + b
                o_v[j, sl] = 1.0 / (1.0 + jnp.exp(-z))
        pltpu.sync_copy(o_v, out_hbm.at[pl.ds(r0, _NCH)])

    return body


_SC_KERNEL = _make_sc_kernel()


def kernel(x, u_table, m_table, fc_w, fc_b):
    xb = x.reshape(_ROWS, _CH, 2).transpose(0, 2, 1).reshape(2 * _ROWS, _CH)
    ut = u_table.reshape(1, -1)
    mt = m_table.reshape(1, -1)
    wb = jnp.concatenate([
        jnp.broadcast_to(fc_w.reshape(1), (_L,)),
        jnp.broadcast_to(fc_b.reshape(1), (_L,)),
    ]).astype(jnp.float32)
    out = _SC_KERNEL(xb, ut, mt, wb)
    return out.reshape(_B, 1)
